# baseline (speedup 1.0000x reference)
import functools

import jax
import jax.numpy as jnp
from jax import lax
from jax.experimental import pallas as pl
from jax.experimental.pallas import tpu as pltpu
from jax.experimental.pallas import tpu_sc as plsc

B, H, D = 16384, 200, 64
NC, NS = 2, 16
NW = NC * NS
BT = 128
NBT = B // BT
BT_W = NBT // NW
HPAD = H + 3
TP = BT + 1
NB = 3
HMAIN = (H // NB) * NB

_mesh = plsc.VectorSubcoreMesh(core_axis_name="c", subcore_axis_name="s")


@functools.partial(
    pl.kernel,
    mesh=_mesh,
    out_type=jax.ShapeDtypeStruct((H, D // 8, NBT, 8, BT), jnp.float32),
    scratch_types=[
        pltpu.VMEM((BT, H), jnp.int32),
        pltpu.VMEM((HPAD, BT), jnp.int32),
        pltpu.VMEM((BT, D), jnp.float32),
        pltpu.VMEM((BT, D), jnp.float32),
        pltpu.VMEM((BT, D), jnp.float32),
        pltpu.VMEM((D, TP), jnp.float32),
        pltpu.VMEM((D, TP), jnp.float32),
        pltpu.VMEM((D, TP), jnp.float32),
        pltpu.SemaphoreType.DMA,
        pltpu.SemaphoreType.DMA,
        pltpu.SemaphoreType.DMA,
        pltpu.SemaphoreType.DMA,
        pltpu.SemaphoreType.DMA,
        pltpu.SemaphoreType.DMA,
    ],
    compiler_params=pltpu.CompilerParams(
        use_tc_tiling_on_sc=False,
        needs_layout_passes=False,
        disable_bounds_checks=True,
    ),
)
def _embed(x_hbm, table_hbm, out_hbm, idx_raw, idx_t, rows0, rows1, rows2,
           trows0, trows1, trows2, gsem0, gsem1, gsem2,
           osem0, osem1, osem2):
    rows_v = [rows0, rows1, rows2]
    trows_v = [trows0, trows1, trows2]
    gsems = [gsem0, gsem1, gsem2]
    osems = [osem0, osem1, osem2]

    wid = lax.axis_index("s") * NC + lax.axis_index("c")
    iota = lax.iota(jnp.int32, 16)
    lane_sb = [iota + sb * 16 for sb in range(8)]
    row_dg = [iota + dg * 16 for dg in range(4)]
    zeros16 = jnp.zeros((16,), jnp.int32)

    def bt_body(k, carry):
        bt = wid * BT_W + k
        b0 = bt * BT

        pltpu.sync_copy(x_hbm.at[pl.ds(b0, BT)], idx_raw)

        def idx_body(h, c):
            hvec = jnp.broadcast_to(h, (16,)).astype(jnp.int32)
            for sb in range(8):
                v = plsc.load_gather(idx_raw, [lane_sb[sb], hvec])
                idx_t[h, pl.ds(sb * 16, 16)] = v
            return c

        lax.fori_loop(0, H, idx_body, 0)
        for sb in range(8):
            for hp in range(H, HPAD):
                idx_t[hp, pl.ds(sb * 16, 16)] = zeros16

        def fire_gather(h, p):
            pltpu.async_copy(
                table_hbm.at[idx_t.at[h]], rows_v[p], gsems[p]
            )

        def wait_gather(p):
            pltpu.make_async_copy(
                table_hbm.at[pl.ds(0, BT)], rows_v[p], gsems[p]
            ).wait()

        def transpose(p):
            rows = rows_v[p]
            trows = trows_v[p]
            for b in range(BT):
                bvec = jnp.full((16,), b, jnp.int32)
                for dg in range(4):
                    v = rows[b, pl.ds(dg * 16, 16)]
                    plsc.store_scatter(trows, [row_dg[dg], bvec], v)

        def fire_store(h, p):
            for dt in range(D // 8):
                pltpu.async_copy(
                    trows_v[p].at[pl.ds(dt * 8, 8), pl.ds(0, BT)],
                    out_hbm.at[h, dt, bt],
                    osems[p],
                )

        def wait_store(p):
            for dt in range(D // 8):
                pltpu.make_async_copy(
                    trows_v[p].at[pl.ds(dt * 8, 8), pl.ds(0, BT)],
                    out_hbm.at[0, dt, 0],
                    osems[p],
                ).wait()

        fire_gather(0, 0)
        fire_gather(1, 1)

        def h_body(i, c):
            for p in range(NB):
                h = NB * i + p

                @pl.when(i > 0)
                def _():
                    wait_store(p)

                wait_gather(p)
                fire_gather(h + 2, (p + 2) % NB)
                transpose(p)
                fire_store(h, p)
            return c

        lax.fori_loop(0, HMAIN // NB, h_body, 0)

        for t in range(H - HMAIN):
            h = HMAIN + t
            p = h % NB
            wait_store(p)
            wait_gather(p)
            fire_gather(h + 2, (p + 2) % NB)
            transpose(p)
            fire_store(h, p)

        for t in range(2):
            wait_gather((H + t) % NB)
        for p in range(NB):
            wait_store(p)
        return carry

    lax.fori_loop(0, BT_W, bt_body, 0)


def kernel(x, weight):
    out5 = _embed(x, weight)
    return jnp.transpose(out5, (2, 4, 0, 1, 3)).reshape(B, H, D)


# speedup vs baseline: 1.0610x; 1.0610x over previous
import functools

import jax
import jax.numpy as jnp
from jax import lax
from jax.experimental import pallas as pl
from jax.experimental.pallas import tpu as pltpu
from jax.experimental.pallas import tpu_sc as plsc

B, H, D = 16384, 200, 64
N = B * H
NC, NS = 2, 16
NW = NC * NS
ROWS_W = B // NW
R = 4
CHUNK = R * H
STEPS = ROWS_W // R
NBUF = 2
NGROUPS = STEPS // NBUF

_mesh = plsc.VectorSubcoreMesh(core_axis_name="c", subcore_axis_name="s")


@functools.partial(
    pl.kernel,
    mesh=_mesh,
    out_type=jax.ShapeDtypeStruct((B, H, D), jnp.float32),
    scratch_types=[
        pltpu.VMEM((CHUNK,), jnp.int32),
        pltpu.VMEM((CHUNK,), jnp.int32),
        pltpu.VMEM((R, H, D), jnp.float32),
        pltpu.VMEM((R, H, D), jnp.float32),
        pltpu.SemaphoreType.DMA,
        pltpu.SemaphoreType.DMA,
        pltpu.SemaphoreType.DMA,
        pltpu.SemaphoreType.DMA,
        pltpu.SemaphoreType.DMA,
        pltpu.SemaphoreType.DMA,
    ],
    compiler_params=pltpu.CompilerParams(
        use_tc_tiling_on_sc=False,
        disable_bounds_checks=True,
    ),
)
def _embed(idx_hbm, table_hbm, out_hbm, idx0, idx1, rows0, rows1,
           gsem0, gsem1, osem0, osem1, isem0, isem1):
    idx_v = [idx0, idx1]
    rows_v = [rows0, rows1]
    gsems = [gsem0, gsem1]
    osems = [osem0, osem1]
    isems = [isem0, isem1]

    wid = lax.axis_index("s") * NC + lax.axis_index("c")
    row_base = wid * ROWS_W

    def fire_idx(chunk_id, b):
        r0 = row_base + chunk_id * R
        pltpu.async_copy(
            idx_hbm.at[pl.ds(r0 * H, CHUNK)], idx_v[b], isems[b]
        )

    def wait_idx(b):
        pltpu.make_async_copy(
            idx_hbm.at[pl.ds(0, CHUNK)], idx_v[b], isems[b]
        ).wait()

    def fire_gathers(b):
        for r in range(R):
            pltpu.async_copy(
                table_hbm.at[idx_v[b].at[pl.ds(r * H, H)]],
                rows_v[b].at[r],
                gsems[b],
            )

    def wait_gathers(b):
        pltpu.make_async_copy(
            out_hbm.at[pl.ds(0, R)], rows_v[b], gsems[b]
        ).wait()

    def fire_store(chunk_id, b):
        r0 = row_base + chunk_id * R
        pltpu.async_copy(rows_v[b], out_hbm.at[pl.ds(r0, R)], osems[b])

    def wait_store(b):
        pltpu.make_async_copy(
            rows_v[b], out_hbm.at[pl.ds(0, R)], osems[b]
        ).wait()

    for b in range(NBUF):
        fire_idx(b, b)
    for b in range(NBUF):
        wait_idx(b)
        fire_gathers(b)

    def group(g0, carry):
        for b in range(NBUF):
            g = g0 * NBUF + b
            wait_gathers(b)
            fire_idx(g + NBUF, b)
            fire_store(g, b)
            wait_store(b)
            wait_idx(b)
            fire_gathers(b)
        return carry

    lax.fori_loop(0, NGROUPS - 1, group, 0)

    for b in range(NBUF):
        g = STEPS - NBUF + b
        wait_gathers(b)
        fire_store(g, b)
    for b in range(NBUF):
        wait_store(b)


def kernel(x, weight):
    flat = x.reshape(-1).astype(jnp.int32)
    return _embed(flat, weight)
